# baseline (speedup 1.0000x reference)
import functools

import jax
import jax.numpy as jnp
from jax import lax
from jax.experimental import pallas as pl
from jax.experimental.pallas import tpu as pltpu
from jax.experimental.pallas import tpu_sc as plsc

N_CORES = 2
N_SUBCORES = 16
N_WORKERS = N_CORES * N_SUBCORES
LANES = 16

CHUNK = 10000


SC_ROWS = 30000
TC_BLOCK = 5000


def _tc_mean3_body(x0_ref, x1_ref, x2_ref, o_ref):
    o_ref[...] = (x0_ref[...] + x1_ref[...] + x2_ref[...]) * (1.0 / 3.0)


def _sc_mean3(total):
    per_worker = total // N_WORKERS
    n_chunks = per_worker // CHUNK
    mesh = plsc.VectorSubcoreMesh(
        core_axis_name="c", subcore_axis_name="s")

    @functools.partial(
        pl.kernel,
        out_type=jax.ShapeDtypeStruct((total,), jnp.float32),
        mesh=mesh,
        scratch_types=[
            pltpu.VMEM((CHUNK,), jnp.float32),
            pltpu.VMEM((CHUNK,), jnp.float32),
            pltpu.VMEM((CHUNK,), jnp.float32),
            pltpu.VMEM((CHUNK,), jnp.float32),
            pltpu.VMEM((CHUNK,), jnp.float32),
            pltpu.VMEM((CHUNK,), jnp.float32),
            pltpu.VMEM((CHUNK,), jnp.float32),
            pltpu.VMEM((CHUNK,), jnp.float32),
            pltpu.SemaphoreType.DMA,
            pltpu.SemaphoreType.DMA,
            pltpu.SemaphoreType.DMA,
            pltpu.SemaphoreType.DMA,
        ],
    )
    def k(x0_hbm, x1_hbm, x2_hbm, out_hbm, a0, a1, b0, b1, c0, c1,
          o0, o1, isem0, isem1, osem0, osem1):
        wid = lax.axis_index("s") * N_CORES + lax.axis_index("c")
        base = wid * per_worker
        a_v, b_v, c_v, o_v = (a0, a1), (b0, b1), (c0, c1), (o0, o1)
        in_sem, out_sem = (isem0, isem1), (osem0, osem1)

        def start_in(i, b):
            off = base + i * CHUNK
            pltpu.async_copy(x0_hbm.at[pl.ds(off, CHUNK)], a_v[b],
                             in_sem[b])
            pltpu.async_copy(x1_hbm.at[pl.ds(off, CHUNK)], b_v[b],
                             in_sem[b])
            pltpu.async_copy(x2_hbm.at[pl.ds(off, CHUNK)], c_v[b],
                             in_sem[b])

        def wait_in(i, b):
            off = base + i * CHUNK
            pltpu.make_async_copy(x0_hbm.at[pl.ds(off, CHUNK)], a_v[b],
                                  in_sem[b]).wait()
            pltpu.make_async_copy(x1_hbm.at[pl.ds(off, CHUNK)], b_v[b],
                                  in_sem[b]).wait()
            pltpu.make_async_copy(x2_hbm.at[pl.ds(off, CHUNK)], c_v[b],
                                  in_sem[b]).wait()

        def start_out(i, b):
            off = base + i * CHUNK
            pltpu.async_copy(o_v[b], out_hbm.at[pl.ds(off, CHUNK)],
                             out_sem[b])

        def wait_out(i, b):
            off = base + i * CHUNK
            pltpu.make_async_copy(o_v[b], out_hbm.at[pl.ds(off, CHUNK)],
                                  out_sem[b]).wait()

        def compute(b):
            av, bv, cv, ov = a_v[b], b_v[b], c_v[b], o_v[b]

            @plsc.parallel_loop(0, CHUNK // LANES, unroll=16)
            def _(j):
                s = pl.ds(j * LANES, LANES)
                ov[s] = (av[s] + bv[s] + cv[s]) * (1.0 / 3.0)

        start_in(0, 0)

        def pair_body(p, _):
            for b in (0, 1):
                i = p * 2 + b

                @pl.when(i + 1 < n_chunks)
                def _():
                    start_in(i + 1, 1 - b)

                wait_in(i, b)

                @pl.when(i >= 2)
                def _():
                    wait_out(i - 2, b)

                compute(b)
                start_out(i, b)
            return 0

        lax.fori_loop(0, n_chunks // 2, pair_body, 0)
        wait_out(n_chunks - 2, 0)
        wait_out(n_chunks - 1, 1)

    return k


def kernel(edge_index, xs_0, xs_1, xs_2):
    n, d = xs_0.shape
    rs = SC_ROWS

    sc_k = _sc_mean3(rs * d)
    sc_out = sc_k(xs_0.reshape(-1), xs_1.reshape(-1), xs_2.reshape(-1))

    off = rs // TC_BLOCK
    in_spec = pl.BlockSpec((TC_BLOCK, d), lambda i: (i + off, 0))
    tc_out = pl.pallas_call(
        _tc_mean3_body,
        grid=((n - rs) // TC_BLOCK,),
        in_specs=[in_spec, in_spec, in_spec],
        out_specs=pl.BlockSpec((TC_BLOCK, d), lambda i: (i, 0)),
        out_shape=jax.ShapeDtypeStruct((n - rs, d), jnp.float32),
    )(xs_0, xs_1, xs_2)

    return jnp.concatenate([sc_out.reshape(rs, d), tc_out], axis=0)


# speedup vs baseline: 1.7826x; 1.7826x over previous
import jax
import jax.numpy as jnp
from jax.experimental import pallas as pl


def _mean3_body(x0_ref, x1_ref, x2_ref, o_ref):
    o_ref[...] = (x0_ref[...] + x1_ref[...] + x2_ref[...]) * (1.0 / 3.0)


def kernel(edge_index, xs_0, xs_1, xs_2):
    n, d = xs_0.shape
    block = 4000
    while n % block != 0:
        block //= 2
    spec = pl.BlockSpec((block, d), lambda i: (i, 0))
    return pl.pallas_call(
        _mean3_body,
        grid=(n // block,),
        in_specs=[spec, spec, spec],
        out_specs=spec,
        out_shape=jax.ShapeDtypeStruct((n, d), xs_0.dtype),
    )(xs_0, xs_1, xs_2)


# speedup vs baseline: 1.7855x; 1.0016x over previous
import jax
import jax.numpy as jnp
from jax.experimental import pallas as pl


def _mean3_body(x0_ref, x1_ref, x2_ref, o_ref):
    o_ref[...] = (x0_ref[...] + x1_ref[...] + x2_ref[...]) * (1.0 / 3.0)


def kernel(edge_index, xs_0, xs_1, xs_2):
    n, d = xs_0.shape
    block = 5000
    while n % block != 0:
        block //= 2
    spec = pl.BlockSpec((block, d), lambda i: (i, 0))
    return pl.pallas_call(
        _mean3_body,
        grid=(n // block,),
        in_specs=[spec, spec, spec],
        out_specs=spec,
        out_shape=jax.ShapeDtypeStruct((n, d), xs_0.dtype),
    )(xs_0, xs_1, xs_2)
